# Initial kernel scaffold; baseline (speedup 1.0000x reference)
import jax
import jax.numpy as jnp
from jax.experimental import pallas as pl


def kernel(encoded, raw_activations):
    raise NotImplementedError("write your pallas kernel here")


# speedup vs baseline: 8.6027x; 8.6027x over previous
import jax
import jax.numpy as jnp
from jax.experimental import pallas as pl
from jax.experimental.pallas import tpu as pltpu

_RB = 8

_SORT8 = [
    (0, 1), (2, 3), (4, 5), (6, 7),
    (0, 2), (1, 3), (4, 6), (5, 7),
    (1, 2), (5, 6),
    (0, 4), (1, 5), (2, 6), (3, 7),
    (2, 4), (3, 5),
    (1, 2), (3, 4), (5, 6),
]
_BITONIC8 = [
    (0, 4), (1, 5), (2, 6), (3, 7),
    (0, 2), (1, 3), (4, 6), (5, 7),
    (0, 1), (2, 3), (4, 5), (6, 7),
]


def _ce(lst, i, j):
    hi = jnp.maximum(lst[i], lst[j])
    lo = jnp.minimum(lst[i], lst[j])
    lst[i] = hi
    lst[j] = lo


def _sort8_desc(vals):
    lst = list(vals)
    for i, j in _SORT8:
        _ce(lst, i, j)
    return lst


def _merge_top8(m, n):
    t = [jnp.maximum(m[i], n[7 - i]) for i in range(8)]
    for i, j in _BITONIC8:
        _ce(t, i, j)
    return t


def _shift_left(x):
    return jnp.concatenate([x[:, 1:], x[:, :1]], axis=1)


def _body(enc_ref, raw_ref, out_ref):
    C = raw_ref.shape[1]
    RB = raw_ref.shape[2]
    Wf = raw_ref.shape[3]
    W = Wf // 2
    WO = W // 2

    blk0 = raw_ref[0, 0:8]
    m = tuple(_sort8_desc([blk0[k] for k in range(8)]))

    def step(i, m):
        blk = raw_ref[0, pl.ds(i * 8, 8)]
        n = _sort8_desc([blk[k] for k in range(8)])
        return tuple(_merge_top8(list(m), n))

    m = jax.lax.fori_loop(1, C // 8, step, m)
    lp = m[7]

    lpT = lp[:, :W]
    lpB = lp[:, W:]
    lpTL = _shift_left(lpT)
    lpBL = _shift_left(lpB)
    hT = lpT >= lpTL
    hB = lpB >= lpBL
    v = jnp.maximum(lpT, lpTL) >= jnp.maximum(lpB, lpBL)

    del WO
    CB = 8
    v3 = jnp.broadcast_to(v[None], (CB, RB, W))
    hT3 = jnp.broadcast_to(hT[None], (CB, RB, W))
    hB3 = jnp.broadcast_to(hB[None], (CB, RB, W))
    for c0 in range(0, C, CB):
        e = enc_ref[0, pl.ds(c0, CB)]
        eT = e[:, :, :W]
        eB = e[:, :, W:]
        eTL = jnp.concatenate([eT[:, :, 1:], eT[:, :, :1]], axis=2)
        eBL = jnp.concatenate([eB[:, :, 1:], eB[:, :, :1]], axis=2)
        sel = jnp.where(v3, jnp.where(hT3, eT, eTL), jnp.where(hB3, eB, eBL))
        out_ref[0, pl.ds(c0, CB)] = sel


def _pooled_call(encoded, raw_activations, interpret=False):
    B, C, H, W = encoded.shape
    Hf, Wf = H // 2, 2 * W
    ef = encoded.reshape(B, C, Hf, Wf)
    rf = raw_activations.reshape(B, C, Hf, Wf)
    out_dup = pl.pallas_call(
        _body,
        grid=(B, Hf // _RB),
        in_specs=[
            pl.BlockSpec((1, C, _RB, Wf), lambda b, h: (b, 0, h, 0)),
            pl.BlockSpec((1, C, _RB, Wf), lambda b, h: (b, 0, h, 0)),
        ],
        out_specs=pl.BlockSpec((1, C, _RB, W), lambda b, h: (b, 0, h, 0)),
        out_shape=jax.ShapeDtypeStruct((B, C, H // 2, W), jnp.float32),
        compiler_params=pltpu.CompilerParams(
            dimension_semantics=("arbitrary", "arbitrary"),
        ),
        interpret=interpret,
    )(ef, rf)
    return out_dup[:, :, :, ::2]


def kernel(encoded, raw_activations):
    return _pooled_call(encoded, raw_activations)


# speedup vs baseline: 9.2379x; 1.0738x over previous
import jax
import jax.numpy as jnp
from jax.experimental import pallas as pl
from jax.experimental.pallas import tpu as pltpu

_RB = 8

_SORT8 = [
    (0, 1), (2, 3), (4, 5), (6, 7),
    (0, 2), (1, 3), (4, 6), (5, 7),
    (1, 2), (5, 6),
    (0, 4), (1, 5), (2, 6), (3, 7),
    (2, 4), (3, 5),
    (1, 2), (3, 4), (5, 6),
]
_BITONIC8 = [
    (0, 4), (1, 5), (2, 6), (3, 7),
    (0, 2), (1, 3), (4, 6), (5, 7),
    (0, 1), (2, 3), (4, 5), (6, 7),
]


def _ce(lst, i, j):
    hi = jnp.maximum(lst[i], lst[j])
    lo = jnp.minimum(lst[i], lst[j])
    lst[i] = hi
    lst[j] = lo


def _sort8_desc(vals):
    lst = list(vals)
    for i, j in _SORT8:
        _ce(lst, i, j)
    return lst


def _merge_top8(m, n):
    t = [jnp.maximum(m[i], n[7 - i]) for i in range(8)]
    for i, j in _BITONIC8:
        _ce(t, i, j)
    return t


def _shift_left(x):
    return jnp.concatenate([x[..., 1:], x[..., :1]], axis=-1)


def _shift_right(x):
    return jnp.concatenate([x[..., -1:], x[..., :-1]], axis=-1)


def _lowest_passing(raw_ref):
    C = raw_ref.shape[1]
    blk0 = raw_ref[0, 0:8]
    m = _sort8_desc([blk0[k] for k in range(8)])
    for i in range(1, C // 8):
        blk = raw_ref[0, i * 8:(i + 1) * 8]
        n = _sort8_desc([blk[k] for k in range(8)])
        m = _merge_top8(m, n)
    return m[7]


def _body(enc_ref, raw_ref, out_ref):
    C = raw_ref.shape[1]
    RB = raw_ref.shape[2]
    W = raw_ref.shape[3] // 2

    lp = _lowest_passing(raw_ref)
    lpT = lp[:, :W]
    lpB = lp[:, W:]

    one = jnp.float32(1.0)
    zero = jnp.float32(0.0)
    lpTL = _shift_left(lpT)
    lpBL = _shift_left(lpB)
    hTf = jnp.where(lpT >= lpTL, one, zero)
    hBf = jnp.where(lpB >= lpBL, one, zero)
    v = jnp.maximum(lpT, lpTL) >= jnp.maximum(lpB, lpBL)
    vf = jnp.where(v, one, zero)
    hf = jnp.where(v, hTf, hBf)

    lane = jax.lax.broadcasted_iota(jnp.int32, (RB, W), 1)
    even = (lane & 1) == 0
    v_dup = jnp.where(even, vf, _shift_right(vf))
    h_dup = jnp.where(even, hf, _shift_right(hf))

    CB = 8
    v3 = jnp.broadcast_to(v_dup[None], (CB, RB, W)) != 0.0
    h3 = jnp.broadcast_to(h_dup[None], (CB, RB, W)) != 0.0
    for c0 in range(0, C, CB):
        cs = pl.ds(c0, CB)
        e = enc_ref[0, cs]
        selA = jnp.where(v3, e[:, :, :W], e[:, :, W:])
        out_ref[0, cs] = jnp.where(h3, selA, _shift_left(selA))


def _pooled_call(encoded, raw_activations, interpret=False):
    B, C, H, W = encoded.shape
    Hf, Wf = H // 2, 2 * W
    ef = encoded.reshape(B, C, Hf, Wf)
    rf = raw_activations.reshape(B, C, Hf, Wf)
    out_dup = pl.pallas_call(
        _body,
        grid=(B, Hf // _RB),
        in_specs=[
            pl.BlockSpec((1, C, _RB, Wf), lambda b, h: (b, 0, h, 0)),
            pl.BlockSpec((1, C, _RB, Wf), lambda b, h: (b, 0, h, 0)),
        ],
        out_specs=pl.BlockSpec((1, C, _RB, W), lambda b, h: (b, 0, h, 0)),
        out_shape=jax.ShapeDtypeStruct((B, C, H // 2, W), jnp.float32),
        compiler_params=pltpu.CompilerParams(
            dimension_semantics=("arbitrary", "arbitrary"),
        ),
        interpret=interpret,
    )(ef, rf)
    return out_dup[:, :, :, ::2]


def kernel(encoded, raw_activations):
    return _pooled_call(encoded, raw_activations)


# speedup vs baseline: 9.4359x; 1.0214x over previous
import jax
import jax.numpy as jnp
from jax.experimental import pallas as pl
from jax.experimental.pallas import tpu as pltpu

_RB = 16

_SORT8 = [
    (0, 1), (2, 3), (4, 5), (6, 7),
    (0, 2), (1, 3), (4, 6), (5, 7),
    (1, 2), (5, 6),
    (0, 4), (1, 5), (2, 6), (3, 7),
    (2, 4), (3, 5),
    (1, 2), (3, 4), (5, 6),
]
_BITONIC8 = [
    (0, 4), (1, 5), (2, 6), (3, 7),
    (0, 2), (1, 3), (4, 6), (5, 7),
    (0, 1), (2, 3), (4, 5), (6, 7),
]


def _ce(lst, i, j):
    hi = jnp.maximum(lst[i], lst[j])
    lo = jnp.minimum(lst[i], lst[j])
    lst[i] = hi
    lst[j] = lo


def _sort8_desc(vals):
    lst = list(vals)
    for i, j in _SORT8:
        _ce(lst, i, j)
    return lst


def _merge_top8(m, n):
    t = [jnp.maximum(m[i], n[7 - i]) for i in range(8)]
    for i, j in _BITONIC8:
        _ce(t, i, j)
    return t


def _shift_left(x):
    return jnp.concatenate([x[..., 1:], x[..., :1]], axis=-1)


def _shift_up(x):
    return jnp.concatenate([x[1:], x[:1]], axis=0)


def _lowest_passing(raw_ref):
    C = raw_ref.shape[1]
    blk0 = raw_ref[0, 0:8]
    m = _sort8_desc([blk0[k] for k in range(8)])
    for i in range(1, C // 8):
        blk = raw_ref[0, i * 8:(i + 1) * 8]
        n = _sort8_desc([blk[k] for k in range(8)])
        m = _merge_top8(m, n)
    return m[7]


def _body(enc_ref, raw_ref, out_ref):
    C = raw_ref.shape[1]
    RB = raw_ref.shape[2]
    W = raw_ref.shape[3]
    RO = RB // 2
    one = jnp.float32(1.0)
    zero = jnp.float32(0.0)

    lp = _lowest_passing(raw_ref)

    lpL = _shift_left(lp)
    hf = jnp.where(lp >= lpL, one, zero)
    rbest = jnp.maximum(lp, lpL)
    vf = jnp.where(rbest >= _shift_up(rbest), one, zero)
    hcf = jnp.where(vf != 0.0, hf, _shift_up(hf))

    ei = jax.lax.broadcasted_iota(jnp.int32, (RO // 2, W), 0) * 2
    vA = jnp.take_along_axis(vf[0:8], ei, axis=0)
    vB = jnp.take_along_axis(vf[8:16], ei, axis=0)
    hA = jnp.take_along_axis(hcf[0:8], ei, axis=0)
    hB = jnp.take_along_axis(hcf[8:16], ei, axis=0)
    hc = jnp.concatenate([hA, hB], axis=0)

    lane = jax.lax.broadcasted_iota(jnp.int32, (RO // 2, W), 1)
    even = (lane & 1) == 0

    def _dup(idx):
        shifted = jnp.concatenate([idx[:, -1:], idx[:, :-1]], axis=1)
        return jnp.where(even, idx, shifted)

    idxA = _dup(ei + jnp.where(vA != 0.0, 0, 1))
    idxB = _dup(ei + jnp.where(vB != 0.0, 0, 1))

    CB = 8
    idxA3 = jnp.broadcast_to(idxA[None], (CB, RO // 2, W))
    idxB3 = jnp.broadcast_to(idxB[None], (CB, RO // 2, W))
    h3 = jnp.broadcast_to(hc[None], (CB, RO, W)) != 0.0
    for c0 in range(0, C, CB):
        cs = pl.ds(c0, CB)
        e = enc_ref[0, cs]
        gA = jnp.take_along_axis(e[:, 0:8], idxA3, axis=1)
        gB = jnp.take_along_axis(e[:, 8:16], idxB3, axis=1)
        selv = jnp.concatenate([gA, gB], axis=1)
        out_ref[0, cs] = jnp.where(h3, selv, _shift_left(selv))


def _pooled_call(encoded, raw_activations, interpret=False):
    B, C, H, W = encoded.shape
    out_dup = pl.pallas_call(
        _body,
        grid=(B, H // _RB),
        in_specs=[
            pl.BlockSpec((1, C, _RB, W), lambda b, h: (b, 0, h, 0)),
            pl.BlockSpec((1, C, _RB, W), lambda b, h: (b, 0, h, 0)),
        ],
        out_specs=pl.BlockSpec((1, C, _RB // 2, W), lambda b, h: (b, 0, h, 0)),
        out_shape=jax.ShapeDtypeStruct((B, C, H // 2, W), jnp.float32),
        compiler_params=pltpu.CompilerParams(
            dimension_semantics=("arbitrary", "arbitrary"),
        ),
        interpret=interpret,
    )(encoded, raw_activations)
    return out_dup[:, :, :, ::2]


def kernel(encoded, raw_activations):
    return _pooled_call(encoded, raw_activations)


# speedup vs baseline: 34.1350x; 3.6176x over previous
import jax
import jax.numpy as jnp
from jax.experimental import pallas as pl
from jax.experimental.pallas import tpu as pltpu

_RB = 16

_SORT8 = [
    (0, 1), (2, 3), (4, 5), (6, 7),
    (0, 2), (1, 3), (4, 6), (5, 7),
    (1, 2), (5, 6),
    (0, 4), (1, 5), (2, 6), (3, 7),
    (2, 4), (3, 5),
    (1, 2), (3, 4), (5, 6),
]
_BITONIC8 = [
    (0, 4), (1, 5), (2, 6), (3, 7),
    (0, 2), (1, 3), (4, 6), (5, 7),
    (0, 1), (2, 3), (4, 5), (6, 7),
]


def _ce(lst, i, j):
    hi = jnp.maximum(lst[i], lst[j])
    lo = jnp.minimum(lst[i], lst[j])
    lst[i] = hi
    lst[j] = lo


def _sort8_desc(vals):
    lst = list(vals)
    for i, j in _SORT8:
        _ce(lst, i, j)
    return lst


def _merge_top8(m, n):
    t = [jnp.maximum(m[i], n[7 - i]) for i in range(8)]
    for i, j in _BITONIC8:
        _ce(t, i, j)
    return t


def _shift_left(x):
    return jnp.concatenate([x[..., 1:], x[..., :1]], axis=-1)


def _shift_up(x):
    return jnp.concatenate([x[1:], x[:1]], axis=0)


def _lowest_passing(raw_ref):
    C = raw_ref.shape[1]
    blk0 = raw_ref[0, 0:8]
    m = _sort8_desc([blk0[k] for k in range(8)])
    for i in range(1, C // 8):
        blk = raw_ref[0, i * 8:(i + 1) * 8]
        n = _sort8_desc([blk[k] for k in range(8)])
        m = _merge_top8(m, n)
    return m[7]


def _body(enc_ref, raw_ref, out_ref):
    C = raw_ref.shape[1]
    RB = raw_ref.shape[2]
    W = raw_ref.shape[3]
    RO = RB // 2
    one = jnp.float32(1.0)
    zero = jnp.float32(0.0)

    lp = _lowest_passing(raw_ref)

    lpL = _shift_left(lp)
    hf = jnp.where(lp >= lpL, one, zero)
    rbest = jnp.maximum(lp, lpL)
    vf = jnp.where(rbest >= _shift_up(rbest), one, zero)
    hcf = jnp.where(vf != 0.0, hf, _shift_up(hf))

    ei = jax.lax.broadcasted_iota(jnp.int32, (RO // 2, W), 0) * 2
    vA = jnp.take_along_axis(vf[0:8], ei, axis=0)
    vB = jnp.take_along_axis(vf[8:16], ei, axis=0)
    hA = jnp.take_along_axis(hcf[0:8], ei, axis=0)
    hB = jnp.take_along_axis(hcf[8:16], ei, axis=0)
    hc = jnp.concatenate([hA, hB], axis=0)

    lane = jax.lax.broadcasted_iota(jnp.int32, (RO // 2, W), 1)
    even = (lane & 1) == 0

    def _dup(idx):
        shifted = jnp.concatenate([idx[:, -1:], idx[:, :-1]], axis=1)
        return jnp.where(even, idx, shifted)

    idxA = _dup(ei + jnp.where(vA != 0.0, 0, 1))
    idxB = _dup(ei + jnp.where(vB != 0.0, 0, 1))

    chunks = [(0, W, 0, W // 2)] if W <= 128 else [(0, 128, 0, 64), (128, W, 64, W // 2)]
    lidx = []
    for (slo, shi, olo, ohi) in chunks:
        ji = jax.lax.broadcasted_iota(jnp.int32, (RO, ohi - olo), 1) * 2
        hcc = jnp.take_along_axis(hc[:, slo:shi], ji, axis=1)
        lidx.append(ji + jnp.where(hcc != 0.0, 0, 1))

    CB = 16
    idxA3 = jnp.broadcast_to(idxA[None], (CB, RO // 2, W))
    idxB3 = jnp.broadcast_to(idxB[None], (CB, RO // 2, W))
    lidx3 = [jnp.broadcast_to(li[None], (CB,) + li.shape) for li in lidx]
    for c0 in range(0, C, CB):
        cs = pl.ds(c0, CB)
        e = enc_ref[0, cs]
        gA = jnp.take_along_axis(e[:, 0:8], idxA3, axis=1)
        gB = jnp.take_along_axis(e[:, 8:16], idxB3, axis=1)
        selv = jnp.concatenate([gA, gB], axis=1)
        for (slo, shi, olo, ohi), li3 in zip(chunks, lidx3):
            out_ref[0, cs, :, olo:ohi] = jnp.take_along_axis(
                selv[:, :, slo:shi], li3, axis=2)


def _pooled_call(encoded, raw_activations, interpret=False):
    B, C, H, W = encoded.shape
    out_dup = pl.pallas_call(
        _body,
        grid=(B, H // _RB),
        in_specs=[
            pl.BlockSpec((1, C, _RB, W), lambda b, h: (b, 0, h, 0)),
            pl.BlockSpec((1, C, _RB, W), lambda b, h: (b, 0, h, 0)),
        ],
        out_specs=pl.BlockSpec((1, C, _RB // 2, W // 2), lambda b, h: (b, 0, h, 0)),
        out_shape=jax.ShapeDtypeStruct((B, C, H // 2, W // 2), jnp.float32),
        compiler_params=pltpu.CompilerParams(
            dimension_semantics=("arbitrary", "arbitrary"),
        ),
        interpret=interpret,
    )(encoded, raw_activations)
    return out_dup


def kernel(encoded, raw_activations):
    return _pooled_call(encoded, raw_activations)
